# baseline (speedup 1.0000x reference)
import jax
import jax.numpy as jnp
from jax import lax
from jax.experimental import pallas as pl
from jax.experimental.pallas import tpu as pltpu
from jax.experimental.pallas import tpu_sc as plsc

N_NODES = 10000
N_TAB = 10240
E_RAW = 320000
E_AUG = E_RAW + N_NODES
NC, NS, LANES = 2, 16, 16
CH = 128
CPT = 164
E_PAD = NS * CPT * CH
ROWS_PER_TILE = N_TAB // NS
H = 8
HC = H // NC
HD = 16
FC = HC * HD


def _qkv_body(x_ref, wq_ref, bq_ref, wk_ref, bk_ref, wv_ref,
              q_ref, k_ref, v_ref):
    xb = x_ref[...]
    q = jnp.dot(xb, wq_ref[0], preferred_element_type=jnp.float32)
    q_ref[0] = jnp.maximum(q + bq_ref[0], 0.0)
    k = jnp.dot(xb, wk_ref[0], preferred_element_type=jnp.float32)
    k_ref[0] = jnp.maximum(k + bk_ref[0], 0.0)
    v_ref[0] = jnp.dot(xb, wv_ref[0], preferred_element_type=jnp.float32)


def _split_cols(w):
    w2 = w.reshape(w.shape[0], NC, FC) if w.ndim == 2 else w.reshape(1, NC, FC)
    return jnp.swapaxes(w2, 0, 1)


def _qkv(x_pad, wq, bq, wk, bk, wv):
    blk = 256
    grid = (N_TAB // blk, NC)
    wspec = pl.BlockSpec((1, 128, FC), lambda i, j: (j, 0, 0))
    bspec = pl.BlockSpec((1, 1, FC), lambda i, j: (j, 0, 0))
    xspec = pl.BlockSpec((blk, 128), lambda i, j: (i, 0))
    ospec = pl.BlockSpec((1, blk, FC), lambda i, j: (j, i, 0))
    out = jax.ShapeDtypeStruct((NC, N_TAB, FC), jnp.float32)
    return pl.pallas_call(
        _qkv_body,
        grid=grid,
        in_specs=[xspec, wspec, bspec, wspec, bspec, wspec],
        out_specs=[ospec, ospec, ospec],
        out_shape=[out, out, out],
    )(x_pad, _split_cols(wq), _split_cols(bq), _split_cols(wk),
      _split_cols(bk), _split_cols(wv))


def _edge_body(q_hbm, k_hbm, v_hbm, ridx_hbm, cidx_hbm, zrow_hbm, zden_hbm,
               out_hbm, den_hbm,
               ridx_s, cidx_s, qb, kb, vb, eb,
               acc_out, acc_den,
               gsem0, gsem1, isem0, isem1, isem2, isem3):
    c = lax.axis_index("c")
    s = lax.axis_index("s")
    lane = lax.broadcasted_iota(jnp.int32, (LANES,), 0)
    zvec = jnp.zeros((LANES,), jnp.float32)
    gsem = (gsem0, gsem1)
    isem = (isem0, isem1, isem2, isem3)

    pltpu.sync_copy(zrow_hbm, acc_out.at[pl.ds(s * ROWS_PER_TILE, ROWS_PER_TILE)])
    pltpu.sync_copy(zden_hbm, acc_den.at[pl.ds(s * ROWS_PER_TILE, ROWS_PER_TILE)])

    pltpu.sync_copy(ridx_hbm.at[s, pl.ds(0, 4)], ridx_s)
    pltpu.sync_copy(cidx_hbm.at[s, pl.ds(0, 4)], cidx_s)

    plsc.subcore_barrier()

    def gather_copies(p, slot):
        return (
            pltpu.make_async_copy(q_hbm.at[c].at[ridx_s.at[slot]], qb.at[p],
                                  gsem[p]),
            pltpu.make_async_copy(k_hbm.at[c].at[cidx_s.at[slot]], kb.at[p],
                                  gsem[p]),
            pltpu.make_async_copy(v_hbm.at[c].at[cidx_s.at[slot]], vb.at[p],
                                  gsem[p]),
        )

    def issue_gather(p, slot):
        pltpu.async_copy(q_hbm.at[c].at[ridx_s.at[slot]], qb.at[p], gsem[p])
        pltpu.async_copy(k_hbm.at[c].at[cidx_s.at[slot]], kb.at[p], gsem[p])
        pltpu.async_copy(v_hbm.at[c].at[cidx_s.at[slot]], vb.at[p], gsem[p])

    idx15 = jnp.full((LANES, 1), HD - 1, jnp.int32)
    gdn = lax.GatherDimensionNumbers(
        offset_dims=(), collapsed_slice_dims=(0,), start_index_map=(0,))

    def _bcast_last(v):
        return lax.gather(v, idx15, gdn, slice_sizes=(1,),
                          mode=lax.GatherScatterMode.PROMISE_IN_BOUNDS)

    def compute_chunk(p, slot):
        qbb, kbb, vbb, ebb = qb.at[p], kb.at[p], vb.at[p], eb.at[p]

        def edge_body(e, carry2):
            row = zvec
            for h in range(HC):
                sl = pl.ds(h * HD, HD)
                cs = plsc.cumsum(qbb[e, sl] * kbb[e, sl])
                w = jnp.exp(_bcast_last(cs))
                row = jnp.where(lane == h, w, row)
                vbb[e, sl] = vbb[e, sl] * w
            ebb[e, :] = row
            return carry2

        lax.fori_loop(0, CH, edge_body, 0, unroll=2)
        pltpu.sync_copy(ebb, acc_den.at[ridx_s.at[slot]], add=True)
        pltpu.sync_copy(vbb, acc_out.at[ridx_s.at[slot]], add=True)

    issue_gather(0, 0)

    def quad_body(t, carry):
        for bb in range(4):
            j = 4 * t + bb
            p = bb % 2
            slot = bb
            nslot = (bb + 1) % 4
            for cp in gather_copies(p, slot):
                cp.wait()
            nxt = j + 1

            @pl.when(jnp.logical_and(nxt >= 4, nxt < CPT))
            def _():
                pltpu.make_async_copy(ridx_hbm.at[s, nxt], ridx_s.at[nslot],
                                      isem[nslot]).wait()
                pltpu.make_async_copy(cidx_hbm.at[s, nxt], cidx_s.at[nslot],
                                      isem[nslot]).wait()

            @pl.when(nxt < CPT)
            def _():
                issue_gather(1 - p, nslot)

            compute_chunk(p, slot)

            @pl.when(j + 4 < CPT)
            def _():
                pltpu.async_copy(ridx_hbm.at[s, j + 4], ridx_s.at[slot],
                                 isem[slot])
                pltpu.async_copy(cidx_hbm.at[s, j + 4], cidx_s.at[slot],
                                 isem[slot])
        return carry

    lax.fori_loop(0, CPT // 4, quad_body, 0)
    plsc.subcore_barrier()
    base = s * ROWS_PER_TILE
    pltpu.sync_copy(acc_out.at[pl.ds(base, ROWS_PER_TILE)],
                    out_hbm.at[c, pl.ds(base, ROWS_PER_TILE)])
    pltpu.sync_copy(acc_den.at[pl.ds(base, ROWS_PER_TILE)],
                    den_hbm.at[c, pl.ds(base, ROWS_PER_TILE)])


def _edge_phase(q, k, v, ridx3, cidx3):
    mesh = plsc.VectorSubcoreMesh(core_axis_name="c", subcore_axis_name="s")
    zrow = jnp.zeros((ROWS_PER_TILE, FC), jnp.float32)
    zden = jnp.zeros((ROWS_PER_TILE, HD), jnp.float32)
    fn = pl.kernel(
        _edge_body,
        out_type=[
            jax.ShapeDtypeStruct((NC, N_TAB, FC), jnp.float32),
            jax.ShapeDtypeStruct((NC, N_TAB, HD), jnp.float32),
        ],
        mesh=mesh,
        compiler_params=pltpu.CompilerParams(
            needs_layout_passes=False, use_tc_tiling_on_sc=False),
        scratch_types=[
            pltpu.VMEM((4, CH), jnp.int32),
            pltpu.VMEM((4, CH), jnp.int32),
            pltpu.VMEM((2, CH, FC), jnp.float32),
            pltpu.VMEM((2, CH, FC), jnp.float32),
            pltpu.VMEM((2, CH, FC), jnp.float32),
            pltpu.VMEM((2, CH, HD), jnp.float32),
            pltpu.VMEM_SHARED((N_TAB, FC), jnp.float32),
            pltpu.VMEM_SHARED((N_TAB, HD), jnp.float32),
            pltpu.SemaphoreType.DMA,
            pltpu.SemaphoreType.DMA,
            pltpu.SemaphoreType.DMA,
            pltpu.SemaphoreType.DMA,
            pltpu.SemaphoreType.DMA,
            pltpu.SemaphoreType.DMA,
        ],
    )
    return fn(q, k, v, ridx3, cidx3, zrow, zden)


def _combine_body(p_ref, d_ref, b_ref, o_ref):
    col_h = lax.broadcasted_iota(jnp.int32, (HC, FC), 1) // HD
    row_h = lax.broadcasted_iota(jnp.int32, (HC, FC), 0)
    expand = (col_h == row_h).astype(jnp.float32)
    halves = []
    for cc in range(NC):
        r = 1.0 / d_ref[cc, :, 0:HC]
        halves.append(
            p_ref[cc]
            * jnp.dot(r, expand, preferred_element_type=jnp.float32))
    o_ref[...] = jnp.concatenate(halves, axis=1) + b_ref[...]


def _combine(parts, dens, bias):
    blk = 400
    grid = (N_NODES // blk,)
    return pl.pallas_call(
        _combine_body,
        grid=grid,
        in_specs=[
            pl.BlockSpec((NC, blk, FC), lambda i: (0, i, 0)),
            pl.BlockSpec((NC, blk, HD), lambda i: (0, i, 0)),
            pl.BlockSpec((1, 128), lambda i: (0, 0)),
        ],
        out_specs=pl.BlockSpec((blk, 128), lambda i: (i, 0)),
        out_shape=jax.ShapeDtypeStruct((N_NODES, 128), jnp.float32),
    )(parts, dens, bias.reshape(1, 128))


@jax.jit
def kernel(x, edge_index, query_kernel, query_bias, key_kernel, key_bias,
           kernel, bias):
    n = x.shape[0]
    x_pad = jnp.concatenate(
        [x, jnp.zeros((N_TAB - n, x.shape[1]), x.dtype)], axis=0)
    q, k, v = _qkv(x_pad, query_kernel, query_bias, key_kernel, key_bias,
                   kernel)

    self_loop = jnp.arange(n, dtype=edge_index.dtype)
    rows = jnp.concatenate([edge_index[0], self_loop])
    cols = jnp.concatenate([edge_index[1], self_loop])
    pad = E_PAD - E_AUG
    dummy = jnp.full((pad,), N_NODES, dtype=rows.dtype)
    ridx3 = jnp.concatenate([rows, dummy]).reshape(NS, CPT, CH)
    cidx3 = jnp.concatenate([cols, dummy]).reshape(NS, CPT, CH)

    outp, denp = _edge_phase(q, k, v, ridx3, cidx3)
    return _combine(outp[:, :N_NODES], denp[:, :N_NODES], bias)


# speedup vs baseline: 2.9236x; 2.9236x over previous
import jax
import jax.numpy as jnp
from jax import lax
from jax.experimental import pallas as pl
from jax.experimental.pallas import tpu as pltpu
from jax.experimental.pallas import tpu_sc as plsc

N_NODES = 10000
N_TAB = 10240
E_RAW = 320000
E_AUG = E_RAW + N_NODES
NC, NS, LANES = 2, 16, 16
CH = 128
CPT = 164
E_PAD = NS * CPT * CH
ROWS_PER_TILE = N_TAB // NS
H = 8
HC = H // NC
HD = 16
FC = HC * HD


def _qkv_body(x_ref, wq_ref, bq_ref, wk_ref, bk_ref, wv_ref,
              q_ref, k_ref, v_ref):
    xb = x_ref[...]
    q = jnp.dot(xb, wq_ref[0], preferred_element_type=jnp.float32)
    q_ref[0] = jnp.maximum(q + bq_ref[0], 0.0)
    k = jnp.dot(xb, wk_ref[0], preferred_element_type=jnp.float32)
    k_ref[0] = jnp.maximum(k + bk_ref[0], 0.0)
    v_ref[0] = jnp.dot(xb, wv_ref[0], preferred_element_type=jnp.float32)


def _split_cols(w):
    w2 = w.reshape(w.shape[0], NC, FC) if w.ndim == 2 else w.reshape(1, NC, FC)
    return jnp.swapaxes(w2, 0, 1)


def _qkv(x_pad, wq, bq, wk, bk, wv):
    blk = 256
    grid = (N_TAB // blk, NC)
    wspec = pl.BlockSpec((1, 128, FC), lambda i, j: (j, 0, 0))
    bspec = pl.BlockSpec((1, 1, FC), lambda i, j: (j, 0, 0))
    xspec = pl.BlockSpec((blk, 128), lambda i, j: (i, 0))
    ospec = pl.BlockSpec((1, blk, FC), lambda i, j: (j, i, 0))
    out = jax.ShapeDtypeStruct((NC, N_TAB, FC), jnp.float32)
    return pl.pallas_call(
        _qkv_body,
        grid=grid,
        in_specs=[xspec, wspec, bspec, wspec, bspec, wspec],
        out_specs=[ospec, ospec, ospec],
        out_shape=[out, out, out],
    )(x_pad, _split_cols(wq), _split_cols(bq), _split_cols(wk),
      _split_cols(bk), _split_cols(wv))


def _edge_body(q_hbm, k_hbm, v_hbm, ridx_hbm, cidx_hbm, zrow_hbm, zden_hbm,
               out_hbm, den_hbm,
               ridx_s, cidx_s, qb, kb, vb, eb,
               acc_out, acc_den,
               gsem0, gsem1, isem0, isem1, isem2, isem3):
    c = lax.axis_index("c")
    s = lax.axis_index("s")
    lane = lax.broadcasted_iota(jnp.int32, (LANES,), 0)
    zvec = jnp.zeros((LANES,), jnp.float32)
    gsem = (gsem0, gsem1)
    isem = (isem0, isem1, isem2, isem3)

    pltpu.sync_copy(zrow_hbm, acc_out.at[pl.ds(s * ROWS_PER_TILE, ROWS_PER_TILE)])
    pltpu.sync_copy(zden_hbm, acc_den.at[pl.ds(s * ROWS_PER_TILE, ROWS_PER_TILE)])

    pltpu.sync_copy(ridx_hbm.at[s, pl.ds(0, 4)], ridx_s)
    pltpu.sync_copy(cidx_hbm.at[s, pl.ds(0, 4)], cidx_s)

    plsc.subcore_barrier()

    def gather_copies(p, slot):
        return (
            pltpu.make_async_copy(q_hbm.at[c].at[ridx_s.at[slot]], qb.at[p],
                                  gsem[p]),
            pltpu.make_async_copy(k_hbm.at[c].at[cidx_s.at[slot]], kb.at[p],
                                  gsem[p]),
            pltpu.make_async_copy(v_hbm.at[c].at[cidx_s.at[slot]], vb.at[p],
                                  gsem[p]),
        )

    def issue_gather(p, slot):
        pltpu.async_copy(q_hbm.at[c].at[ridx_s.at[slot]], qb.at[p], gsem[p])
        pltpu.async_copy(k_hbm.at[c].at[cidx_s.at[slot]], kb.at[p], gsem[p])
        pltpu.async_copy(v_hbm.at[c].at[cidx_s.at[slot]], vb.at[p], gsem[p])

    idx15 = jnp.full((LANES, 1), HD - 1, jnp.int32)
    gdn = lax.GatherDimensionNumbers(
        offset_dims=(), collapsed_slice_dims=(0,), start_index_map=(0,))

    def _bcast_last(v):
        return lax.gather(v, idx15, gdn, slice_sizes=(1,),
                          mode=lax.GatherScatterMode.PROMISE_IN_BOUNDS)

    def compute_chunk(p, slot):
        qbb, kbb, vbb, ebb = qb.at[p], kb.at[p], vb.at[p], eb.at[p]

        EB = 4

        def batch_body(t2, carry2):
            base = t2 * EB
            prods = []
            for i in range(EB):
                e = base + i
                for h in range(HC):
                    sl = pl.ds(h * HD, HD)
                    prods.append(qbb[e, sl] * kbb[e, sl])
            css = [plsc.cumsum(pr) for pr in prods]
            ws = [jnp.exp(_bcast_last(cs)) for cs in css]
            for i in range(EB):
                e = base + i
                row = zvec
                for h in range(HC):
                    w = ws[i * HC + h]
                    sl = pl.ds(h * HD, HD)
                    vbb[e, sl] = vbb[e, sl] * w
                    row = jnp.where(lane == h, w, row)
                ebb[e, :] = row
            return carry2

        lax.fori_loop(0, CH // EB, batch_body, 0)
        pltpu.sync_copy(ebb, acc_den.at[ridx_s.at[slot]], add=True)
        pltpu.sync_copy(vbb, acc_out.at[ridx_s.at[slot]], add=True)

    issue_gather(0, 0)

    def quad_body(t, carry):
        for bb in range(4):
            j = 4 * t + bb
            p = bb % 2
            slot = bb
            nslot = (bb + 1) % 4
            for cp in gather_copies(p, slot):
                cp.wait()
            nxt = j + 1

            @pl.when(jnp.logical_and(nxt >= 4, nxt < CPT))
            def _():
                pltpu.make_async_copy(ridx_hbm.at[s, nxt], ridx_s.at[nslot],
                                      isem[nslot]).wait()
                pltpu.make_async_copy(cidx_hbm.at[s, nxt], cidx_s.at[nslot],
                                      isem[nslot]).wait()

            @pl.when(nxt < CPT)
            def _():
                issue_gather(1 - p, nslot)

            compute_chunk(p, slot)

            @pl.when(j + 4 < CPT)
            def _():
                pltpu.async_copy(ridx_hbm.at[s, j + 4], ridx_s.at[slot],
                                 isem[slot])
                pltpu.async_copy(cidx_hbm.at[s, j + 4], cidx_s.at[slot],
                                 isem[slot])
        return carry

    lax.fori_loop(0, CPT // 4, quad_body, 0)
    plsc.subcore_barrier()
    base = s * ROWS_PER_TILE
    pltpu.sync_copy(acc_out.at[pl.ds(base, ROWS_PER_TILE)],
                    out_hbm.at[c, pl.ds(base, ROWS_PER_TILE)])
    pltpu.sync_copy(acc_den.at[pl.ds(base, ROWS_PER_TILE)],
                    den_hbm.at[c, pl.ds(base, ROWS_PER_TILE)])


def _edge_phase(q, k, v, ridx3, cidx3):
    mesh = plsc.VectorSubcoreMesh(core_axis_name="c", subcore_axis_name="s")
    zrow = jnp.zeros((ROWS_PER_TILE, FC), jnp.float32)
    zden = jnp.zeros((ROWS_PER_TILE, HD), jnp.float32)
    fn = pl.kernel(
        _edge_body,
        out_type=[
            jax.ShapeDtypeStruct((NC, N_TAB, FC), jnp.float32),
            jax.ShapeDtypeStruct((NC, N_TAB, HD), jnp.float32),
        ],
        mesh=mesh,
        compiler_params=pltpu.CompilerParams(
            needs_layout_passes=False, use_tc_tiling_on_sc=False),
        scratch_types=[
            pltpu.VMEM((4, CH), jnp.int32),
            pltpu.VMEM((4, CH), jnp.int32),
            pltpu.VMEM((2, CH, FC), jnp.float32),
            pltpu.VMEM((2, CH, FC), jnp.float32),
            pltpu.VMEM((2, CH, FC), jnp.float32),
            pltpu.VMEM((2, CH, HD), jnp.float32),
            pltpu.VMEM_SHARED((N_TAB, FC), jnp.float32),
            pltpu.VMEM_SHARED((N_TAB, HD), jnp.float32),
            pltpu.SemaphoreType.DMA,
            pltpu.SemaphoreType.DMA,
            pltpu.SemaphoreType.DMA,
            pltpu.SemaphoreType.DMA,
            pltpu.SemaphoreType.DMA,
            pltpu.SemaphoreType.DMA,
        ],
    )
    return fn(q, k, v, ridx3, cidx3, zrow, zden)


def _combine_body(p_ref, d_ref, b_ref, o_ref):
    col_h = lax.broadcasted_iota(jnp.int32, (HC, FC), 1) // HD
    row_h = lax.broadcasted_iota(jnp.int32, (HC, FC), 0)
    expand = (col_h == row_h).astype(jnp.float32)
    halves = []
    for cc in range(NC):
        r = 1.0 / d_ref[cc, :, 0:HC]
        halves.append(
            p_ref[cc]
            * jnp.dot(r, expand, preferred_element_type=jnp.float32))
    o_ref[...] = jnp.concatenate(halves, axis=1) + b_ref[...]


def _combine(parts, dens, bias):
    blk = 400
    grid = (N_NODES // blk,)
    return pl.pallas_call(
        _combine_body,
        grid=grid,
        in_specs=[
            pl.BlockSpec((NC, blk, FC), lambda i: (0, i, 0)),
            pl.BlockSpec((NC, blk, HD), lambda i: (0, i, 0)),
            pl.BlockSpec((1, 128), lambda i: (0, 0)),
        ],
        out_specs=pl.BlockSpec((blk, 128), lambda i: (i, 0)),
        out_shape=jax.ShapeDtypeStruct((N_NODES, 128), jnp.float32),
    )(parts, dens, bias.reshape(1, 128))


@jax.jit
def kernel(x, edge_index, query_kernel, query_bias, key_kernel, key_bias,
           kernel, bias):
    n = x.shape[0]
    x_pad = jnp.concatenate(
        [x, jnp.zeros((N_TAB - n, x.shape[1]), x.dtype)], axis=0)
    q, k, v = _qkv(x_pad, query_kernel, query_bias, key_kernel, key_bias,
                   kernel)

    self_loop = jnp.arange(n, dtype=edge_index.dtype)
    rows = jnp.concatenate([edge_index[0], self_loop])
    cols = jnp.concatenate([edge_index[1], self_loop])
    pad = E_PAD - E_AUG
    dummy = jnp.full((pad,), N_NODES, dtype=rows.dtype)
    ridx3 = jnp.concatenate([rows, dummy]).reshape(NS, CPT, CH)
    cidx3 = jnp.concatenate([cols, dummy]).reshape(NS, CPT, CH)

    outp, denp = _edge_phase(q, k, v, ridx3, cidx3)
    return _combine(outp[:, :N_NODES], denp[:, :N_NODES], bias)


# speedup vs baseline: 2.9309x; 1.0025x over previous
import jax
import jax.numpy as jnp
from jax import lax
from jax.experimental import pallas as pl
from jax.experimental.pallas import tpu as pltpu
from jax.experimental.pallas import tpu_sc as plsc

N_NODES = 10000
N_TAB = 10240
E_RAW = 320000
E_AUG = E_RAW + N_NODES
NC, NS, LANES = 2, 16, 16
CH = 128
CPT = 164
E_PAD = NS * CPT * CH
ROWS_PER_TILE = N_TAB // NS
H = 8
HC = H // NC
HD = 16
FC = HC * HD


def _qkv_body(x_ref, wq_ref, bq_ref, wk_ref, bk_ref, wv_ref,
              q_ref, k_ref, v_ref):
    xb = x_ref[...]
    q = jnp.dot(xb, wq_ref[0], preferred_element_type=jnp.float32)
    q_ref[0] = jnp.maximum(q + bq_ref[0], 0.0)
    k = jnp.dot(xb, wk_ref[0], preferred_element_type=jnp.float32)
    k_ref[0] = jnp.maximum(k + bk_ref[0], 0.0)
    v_ref[0] = jnp.dot(xb, wv_ref[0], preferred_element_type=jnp.float32)


def _split_cols(w):
    w2 = w.reshape(w.shape[0], NC, FC) if w.ndim == 2 else w.reshape(1, NC, FC)
    return jnp.swapaxes(w2, 0, 1)


def _qkv(x_pad, wq, bq, wk, bk, wv):
    blk = 256
    grid = (N_TAB // blk, NC)
    wspec = pl.BlockSpec((1, 128, FC), lambda i, j: (j, 0, 0))
    bspec = pl.BlockSpec((1, 1, FC), lambda i, j: (j, 0, 0))
    xspec = pl.BlockSpec((blk, 128), lambda i, j: (i, 0))
    ospec = pl.BlockSpec((1, blk, FC), lambda i, j: (j, i, 0))
    out = jax.ShapeDtypeStruct((NC, N_TAB, FC), jnp.float32)
    return pl.pallas_call(
        _qkv_body,
        grid=grid,
        in_specs=[xspec, wspec, bspec, wspec, bspec, wspec],
        out_specs=[ospec, ospec, ospec],
        out_shape=[out, out, out],
    )(x_pad, _split_cols(wq), _split_cols(bq), _split_cols(wk),
      _split_cols(bk), _split_cols(wv))


def _edge_body(q_hbm, k_hbm, v_hbm, ridx_hbm, cidx_hbm, zrow_hbm, zden_hbm,
               bias_hbm,
               out_hbm,
               ridx_s, cidx_s, qb, kb, vb, eb, bias_v,
               acc_out, acc_den,
               gsem0, gsem1, isem0, isem1, isem2, isem3):
    c = lax.axis_index("c")
    s = lax.axis_index("s")
    lane = lax.broadcasted_iota(jnp.int32, (LANES,), 0)
    zvec = jnp.zeros((LANES,), jnp.float32)
    gsem = (gsem0, gsem1)
    isem = (isem0, isem1, isem2, isem3)

    pltpu.sync_copy(zrow_hbm, acc_out.at[pl.ds(s * ROWS_PER_TILE, ROWS_PER_TILE)])
    pltpu.sync_copy(zden_hbm, acc_den.at[pl.ds(s * ROWS_PER_TILE, ROWS_PER_TILE)])

    pltpu.sync_copy(ridx_hbm.at[s, pl.ds(0, 4)], ridx_s)
    pltpu.sync_copy(cidx_hbm.at[s, pl.ds(0, 4)], cidx_s)

    plsc.subcore_barrier()

    def gather_copies(p, slot):
        return (
            pltpu.make_async_copy(q_hbm.at[c].at[ridx_s.at[slot]], qb.at[p],
                                  gsem[p]),
            pltpu.make_async_copy(k_hbm.at[c].at[cidx_s.at[slot]], kb.at[p],
                                  gsem[p]),
            pltpu.make_async_copy(v_hbm.at[c].at[cidx_s.at[slot]], vb.at[p],
                                  gsem[p]),
        )

    def issue_gather(p, slot):
        pltpu.async_copy(q_hbm.at[c].at[ridx_s.at[slot]], qb.at[p], gsem[p])
        pltpu.async_copy(k_hbm.at[c].at[cidx_s.at[slot]], kb.at[p], gsem[p])
        pltpu.async_copy(v_hbm.at[c].at[cidx_s.at[slot]], vb.at[p], gsem[p])

    idx15 = jnp.full((LANES, 1), HD - 1, jnp.int32)
    gdn = lax.GatherDimensionNumbers(
        offset_dims=(), collapsed_slice_dims=(0,), start_index_map=(0,))

    def _bcast_last(v):
        return lax.gather(v, idx15, gdn, slice_sizes=(1,),
                          mode=lax.GatherScatterMode.PROMISE_IN_BOUNDS)

    def _bcast_lane(v, h):
        idx = jnp.full((LANES, 1), h, jnp.int32)
        return lax.gather(v, idx, gdn, slice_sizes=(1,),
                          mode=lax.GatherScatterMode.PROMISE_IN_BOUNDS)

    def compute_chunk(p, slot):
        qbb, kbb, vbb, ebb = qb.at[p], kb.at[p], vb.at[p], eb.at[p]

        EB = 4

        def batch_body(t2, carry2):
            base = t2 * EB
            prods = []
            for i in range(EB):
                e = base + i
                for h in range(HC):
                    sl = pl.ds(h * HD, HD)
                    prods.append(qbb[e, sl] * kbb[e, sl])
            css = [plsc.cumsum(pr) for pr in prods]
            ws = [jnp.exp(_bcast_last(cs)) for cs in css]
            for i in range(EB):
                e = base + i
                row = zvec
                for h in range(HC):
                    w = ws[i * HC + h]
                    sl = pl.ds(h * HD, HD)
                    vbb[e, sl] = vbb[e, sl] * w
                    row = jnp.where(lane == h, w, row)
                ebb[e, :] = row
            return carry2

        lax.fori_loop(0, CH // EB, batch_body, 0)
        pltpu.sync_copy(ebb, acc_den.at[ridx_s.at[slot]], add=True)
        pltpu.sync_copy(vbb, acc_out.at[ridx_s.at[slot]], add=True)

    issue_gather(0, 0)

    def quad_body(t, carry):
        for bb in range(4):
            j = 4 * t + bb
            p = bb % 2
            slot = bb
            nslot = (bb + 1) % 4
            for cp in gather_copies(p, slot):
                cp.wait()
            nxt = j + 1

            @pl.when(jnp.logical_and(nxt >= 4, nxt < CPT))
            def _():
                pltpu.make_async_copy(ridx_hbm.at[s, nxt], ridx_s.at[nslot],
                                      isem[nslot]).wait()
                pltpu.make_async_copy(cidx_hbm.at[s, nxt], cidx_s.at[nslot],
                                      isem[nslot]).wait()

            @pl.when(nxt < CPT)
            def _():
                issue_gather(1 - p, nslot)

            compute_chunk(p, slot)

            @pl.when(j + 4 < CPT)
            def _():
                pltpu.async_copy(ridx_hbm.at[s, j + 4], ridx_s.at[slot],
                                 isem[slot])
                pltpu.async_copy(cidx_hbm.at[s, j + 4], cidx_s.at[slot],
                                 isem[slot])
        return carry

    lax.fori_loop(0, CPT // 4, quad_body, 0)
    plsc.subcore_barrier()

    pltpu.sync_copy(bias_hbm.at[c], bias_v)
    bias_h = [bias_v[pl.ds(h * HD, HD)] for h in range(HC)]

    def blk_body(bi, carry):
        r0 = s * ROWS_PER_TILE + bi * CH
        pltpu.sync_copy(acc_out.at[pl.ds(r0, CH)], qb.at[0])
        pltpu.sync_copy(acc_den.at[pl.ds(r0, CH)], eb.at[0])

        def row_body(e, carry2):
            rcp = 1.0 / eb[0, e, :]
            for h in range(HC):
                sl = pl.ds(h * HD, HD)
                qb[0, e, sl] = qb[0, e, sl] * _bcast_lane(rcp, h) + bias_h[h]
            return carry2

        lax.fori_loop(0, CH, row_body, 0, unroll=2)
        pltpu.sync_copy(qb.at[0],
                        out_hbm.at[pl.ds(r0, CH), pl.ds(c * FC, FC)])
        return carry

    lax.fori_loop(0, ROWS_PER_TILE // CH, blk_body, 0)


def _edge_phase(q, k, v, ridx3, cidx3, bias2):
    mesh = plsc.VectorSubcoreMesh(core_axis_name="c", subcore_axis_name="s")
    zrow = jnp.zeros((ROWS_PER_TILE, FC), jnp.float32)
    zden = jnp.zeros((ROWS_PER_TILE, HD), jnp.float32)
    fn = pl.kernel(
        _edge_body,
        out_type=jax.ShapeDtypeStruct((N_TAB, 128), jnp.float32),
        mesh=mesh,
        compiler_params=pltpu.CompilerParams(
            needs_layout_passes=False, use_tc_tiling_on_sc=False),
        scratch_types=[
            pltpu.VMEM((4, CH), jnp.int32),
            pltpu.VMEM((4, CH), jnp.int32),
            pltpu.VMEM((2, CH, FC), jnp.float32),
            pltpu.VMEM((2, CH, FC), jnp.float32),
            pltpu.VMEM((2, CH, FC), jnp.float32),
            pltpu.VMEM((2, CH, HD), jnp.float32),
            pltpu.VMEM((FC,), jnp.float32),
            pltpu.VMEM_SHARED((N_TAB, FC), jnp.float32),
            pltpu.VMEM_SHARED((N_TAB, HD), jnp.float32),
            pltpu.SemaphoreType.DMA,
            pltpu.SemaphoreType.DMA,
            pltpu.SemaphoreType.DMA,
            pltpu.SemaphoreType.DMA,
            pltpu.SemaphoreType.DMA,
            pltpu.SemaphoreType.DMA,
        ],
    )
    return fn(q, k, v, ridx3, cidx3, zrow, zden, bias2)


def _combine_body(p_ref, d_ref, b_ref, o_ref):
    col_h = lax.broadcasted_iota(jnp.int32, (HC, FC), 1) // HD
    row_h = lax.broadcasted_iota(jnp.int32, (HC, FC), 0)
    expand = (col_h == row_h).astype(jnp.float32)
    halves = []
    for cc in range(NC):
        r = 1.0 / d_ref[cc, :, 0:HC]
        halves.append(
            p_ref[cc]
            * jnp.dot(r, expand, preferred_element_type=jnp.float32))
    o_ref[...] = jnp.concatenate(halves, axis=1) + b_ref[...]


def _combine(parts, dens, bias):
    blk = 400
    grid = (N_NODES // blk,)
    return pl.pallas_call(
        _combine_body,
        grid=grid,
        in_specs=[
            pl.BlockSpec((NC, blk, FC), lambda i: (0, i, 0)),
            pl.BlockSpec((NC, blk, HD), lambda i: (0, i, 0)),
            pl.BlockSpec((1, 128), lambda i: (0, 0)),
        ],
        out_specs=pl.BlockSpec((blk, 128), lambda i: (i, 0)),
        out_shape=jax.ShapeDtypeStruct((N_NODES, 128), jnp.float32),
    )(parts, dens, bias.reshape(1, 128))


@jax.jit
def kernel(x, edge_index, query_kernel, query_bias, key_kernel, key_bias,
           kernel, bias):
    n = x.shape[0]
    x_pad = jnp.concatenate(
        [x, jnp.zeros((N_TAB - n, x.shape[1]), x.dtype)], axis=0)
    q, k, v = _qkv(x_pad, query_kernel, query_bias, key_kernel, key_bias,
                   kernel)

    self_loop = jnp.arange(n, dtype=edge_index.dtype)
    rows = jnp.concatenate([edge_index[0], self_loop])
    cols = jnp.concatenate([edge_index[1], self_loop])
    pad = E_PAD - E_AUG
    dummy = jnp.full((pad,), N_NODES, dtype=rows.dtype)
    ridx3 = jnp.concatenate([rows, dummy]).reshape(NS, CPT, CH)
    cidx3 = jnp.concatenate([cols, dummy]).reshape(NS, CPT, CH)

    bias2 = bias.reshape(NC, FC)
    outp = _edge_phase(q, k, v, ridx3, cidx3, bias2)
    return outp[:N_NODES]


# speedup vs baseline: 2.9326x; 1.0006x over previous
import jax
import jax.numpy as jnp
from jax import lax
from jax.experimental import pallas as pl
from jax.experimental.pallas import tpu as pltpu
from jax.experimental.pallas import tpu_sc as plsc

N_NODES = 10000
N_TAB = 10240
E_RAW = 320000
E_AUG = E_RAW + N_NODES
NC, NS, LANES = 2, 16, 16
CH = 128
CPT = 164
E_PAD = NS * CPT * CH
ROWS_PER_TILE = N_TAB // NS
H = 8
HC = H // NC
HD = 16
FC = HC * HD


def _qkv_body(x_ref, wq_ref, bq_ref, wk_ref, bk_ref, wv_ref,
              q_ref, k_ref, v_ref):
    xb = x_ref[...]
    q = jnp.dot(xb, wq_ref[0], preferred_element_type=jnp.float32)
    q_ref[0] = jnp.maximum(q + bq_ref[0], 0.0)
    k = jnp.dot(xb, wk_ref[0], preferred_element_type=jnp.float32)
    k_ref[0] = jnp.maximum(k + bk_ref[0], 0.0)
    v_ref[0] = jnp.dot(xb, wv_ref[0], preferred_element_type=jnp.float32)


def _split_cols(w):
    w2 = w.reshape(w.shape[0], NC, FC) if w.ndim == 2 else w.reshape(1, NC, FC)
    return jnp.swapaxes(w2, 0, 1)


def _qkv(x_pad, wq, bq, wk, bk, wv):
    blk = 256
    grid = (N_TAB // blk, NC)
    wspec = pl.BlockSpec((1, 128, FC), lambda i, j: (j, 0, 0))
    bspec = pl.BlockSpec((1, 1, FC), lambda i, j: (j, 0, 0))
    xspec = pl.BlockSpec((blk, 128), lambda i, j: (i, 0))
    ospec = pl.BlockSpec((1, blk, FC), lambda i, j: (j, i, 0))
    out = jax.ShapeDtypeStruct((NC, N_TAB, FC), jnp.float32)
    return pl.pallas_call(
        _qkv_body,
        grid=grid,
        in_specs=[xspec, wspec, bspec, wspec, bspec, wspec],
        out_specs=[ospec, ospec, ospec],
        out_shape=[out, out, out],
    )(x_pad, _split_cols(wq), _split_cols(bq), _split_cols(wk),
      _split_cols(bk), _split_cols(wv))


def _edge_body(q_hbm, k_hbm, v_hbm, ridx_hbm, cidx_hbm, zrow_hbm, zden_hbm,
               bias_hbm,
               out_hbm,
               ridx_s, cidx_s, qb, kb, vb, eb, bias_v,
               acc_out, acc_den,
               gsem0, gsem1, isem0, isem1, isem2, isem3):
    c = lax.axis_index("c")
    s = lax.axis_index("s")
    lane = lax.broadcasted_iota(jnp.int32, (LANES,), 0)
    zvec = jnp.zeros((LANES,), jnp.float32)
    gsem = (gsem0, gsem1)
    isem = (isem0, isem1, isem2, isem3)

    pltpu.sync_copy(zrow_hbm, acc_out.at[pl.ds(s * ROWS_PER_TILE, ROWS_PER_TILE)])
    pltpu.sync_copy(zden_hbm, acc_den.at[pl.ds(s * ROWS_PER_TILE, ROWS_PER_TILE)])

    pltpu.sync_copy(ridx_hbm.at[s, pl.ds(0, 4)], ridx_s)
    pltpu.sync_copy(cidx_hbm.at[s, pl.ds(0, 4)], cidx_s)

    plsc.subcore_barrier()

    def gather_copies(p, slot):
        return (
            pltpu.make_async_copy(q_hbm.at[c].at[ridx_s.at[slot]], qb.at[p],
                                  gsem[p]),
            pltpu.make_async_copy(k_hbm.at[c].at[cidx_s.at[slot]], kb.at[p],
                                  gsem[p]),
            pltpu.make_async_copy(v_hbm.at[c].at[cidx_s.at[slot]], vb.at[p],
                                  gsem[p]),
        )

    def issue_gather(p, slot):
        pltpu.async_copy(q_hbm.at[c].at[ridx_s.at[slot]], qb.at[p], gsem[p])
        pltpu.async_copy(k_hbm.at[c].at[cidx_s.at[slot]], kb.at[p], gsem[p])
        pltpu.async_copy(v_hbm.at[c].at[cidx_s.at[slot]], vb.at[p], gsem[p])

    idx15 = jnp.full((LANES, 1), HD - 1, jnp.int32)
    gdn = lax.GatherDimensionNumbers(
        offset_dims=(), collapsed_slice_dims=(0,), start_index_map=(0,))

    def _bcast_last(v):
        return lax.gather(v, idx15, gdn, slice_sizes=(1,),
                          mode=lax.GatherScatterMode.PROMISE_IN_BOUNDS)

    def _bcast_lane(v, h):
        idx = jnp.full((LANES, 1), h, jnp.int32)
        return lax.gather(v, idx, gdn, slice_sizes=(1,),
                          mode=lax.GatherScatterMode.PROMISE_IN_BOUNDS)

    def compute_chunk(p, slot):
        qbb, kbb, vbb, ebb = qb.at[p], kb.at[p], vb.at[p], eb.at[p]

        EB = 4

        def batch_body(t2, carry2):
            base = t2 * EB
            prods = []
            for i in range(EB):
                e = base + i
                for h in range(HC):
                    sl = pl.ds(h * HD, HD)
                    prods.append(qbb[e, sl] * kbb[e, sl])
            css = [plsc.cumsum(pr) for pr in prods]
            ws = [jnp.exp(_bcast_last(cs)) for cs in css]
            for i in range(EB):
                e = base + i
                row = zvec
                for h in range(HC):
                    w = ws[i * HC + h]
                    sl = pl.ds(h * HD, HD)
                    vbb[e, sl] = vbb[e, sl] * w
                    row = jnp.where(lane == h, w, row)
                ebb[e, :] = row
            return carry2

        lax.fori_loop(0, CH // EB, batch_body, 0)
        pltpu.sync_copy(ebb, acc_den.at[ridx_s.at[slot]], add=True)
        pltpu.sync_copy(vbb, acc_out.at[ridx_s.at[slot]], add=True)

    issue_gather(0, 0)

    def quad_body(t, carry):
        for bb in range(4):
            j = 4 * t + bb
            p = bb % 2
            slot = bb
            nslot = (bb + 1) % 4
            for cp in gather_copies(p, slot):
                cp.wait()
            nxt = j + 1

            @pl.when(jnp.logical_and(nxt >= 4, nxt < CPT))
            def _():
                pltpu.make_async_copy(ridx_hbm.at[s, nxt], ridx_s.at[nslot],
                                      isem[nslot]).wait()
                pltpu.make_async_copy(cidx_hbm.at[s, nxt], cidx_s.at[nslot],
                                      isem[nslot]).wait()

            @pl.when(nxt < CPT)
            def _():
                issue_gather(1 - p, nslot)

            compute_chunk(p, slot)

            @pl.when(j + 4 < CPT)
            def _():
                pltpu.async_copy(ridx_hbm.at[s, j + 4], ridx_s.at[slot],
                                 isem[slot])
                pltpu.async_copy(cidx_hbm.at[s, j + 4], cidx_s.at[slot],
                                 isem[slot])
        return carry

    lax.fori_loop(0, CPT // 4, quad_body, 0)
    plsc.subcore_barrier()

    pltpu.sync_copy(bias_hbm.at[c], bias_v)
    bias_h = [bias_v[pl.ds(h * HD, HD)] for h in range(HC)]

    def blk_body(bi, carry):
        r0 = s * ROWS_PER_TILE + bi * CH
        pltpu.sync_copy(acc_out.at[pl.ds(r0, CH)], qb.at[0])
        pltpu.sync_copy(acc_den.at[pl.ds(r0, CH)], eb.at[0])

        def row_body(e, carry2):
            rcp = 1.0 / eb[0, e, :]
            for h in range(HC):
                sl = pl.ds(h * HD, HD)
                qb[0, e, sl] = qb[0, e, sl] * _bcast_lane(rcp, h) + bias_h[h]
            return carry2

        lax.fori_loop(0, CH, row_body, 0, unroll=2)
        pltpu.sync_copy(qb.at[0],
                        out_hbm.at[pl.ds(r0, CH), pl.ds(c * FC, FC)])
        return carry

    lax.fori_loop(0, ROWS_PER_TILE // CH, blk_body, 0)


def _edge_phase(q, k, v, ridx3, cidx3, bias2):
    mesh = plsc.VectorSubcoreMesh(core_axis_name="c", subcore_axis_name="s")
    zrow = jnp.zeros((ROWS_PER_TILE, FC), jnp.float32)
    zden = jnp.zeros((ROWS_PER_TILE, HD), jnp.float32)
    fn = pl.kernel(
        _edge_body,
        out_type=jax.ShapeDtypeStruct((N_TAB, 128), jnp.float32),
        mesh=mesh,
        compiler_params=pltpu.CompilerParams(
            needs_layout_passes=False, use_tc_tiling_on_sc=False),
        scratch_types=[
            pltpu.VMEM((4, CH), jnp.int32),
            pltpu.VMEM((4, CH), jnp.int32),
            pltpu.VMEM((2, CH, FC), jnp.float32),
            pltpu.VMEM((2, CH, FC), jnp.float32),
            pltpu.VMEM((2, CH, FC), jnp.float32),
            pltpu.VMEM((2, CH, HD), jnp.float32),
            pltpu.VMEM((FC,), jnp.float32),
            pltpu.VMEM_SHARED((N_TAB, FC), jnp.float32),
            pltpu.VMEM_SHARED((N_TAB, HD), jnp.float32),
            pltpu.SemaphoreType.DMA,
            pltpu.SemaphoreType.DMA,
            pltpu.SemaphoreType.DMA,
            pltpu.SemaphoreType.DMA,
            pltpu.SemaphoreType.DMA,
            pltpu.SemaphoreType.DMA,
        ],
    )
    return fn(q, k, v, ridx3, cidx3, zrow, zden, bias2)


@jax.jit
def kernel(x, edge_index, query_kernel, query_bias, key_kernel, key_bias,
           kernel, bias):
    n = x.shape[0]
    x_pad = jnp.concatenate(
        [x, jnp.zeros((N_TAB - n, x.shape[1]), x.dtype)], axis=0)
    q, k, v = _qkv(x_pad, query_kernel, query_bias, key_kernel, key_bias,
                   kernel)

    self_loop = jnp.arange(n, dtype=edge_index.dtype)
    rows = jnp.concatenate([edge_index[0], self_loop])
    cols = jnp.concatenate([edge_index[1], self_loop])
    pad = E_PAD - E_AUG
    dummy = jnp.full((pad,), N_NODES, dtype=rows.dtype)
    ridx3 = jnp.concatenate([rows, dummy]).reshape(NS, CPT, CH)
    cidx3 = jnp.concatenate([cols, dummy]).reshape(NS, CPT, CH)

    bias2 = bias.reshape(NC, FC)
    outp = _edge_phase(q, k, v, ridx3, cidx3, bias2)
    return outp[:N_NODES]


# speedup vs baseline: 2.9663x; 1.0115x over previous
import jax
import jax.numpy as jnp
from jax import lax
from jax.experimental import pallas as pl
from jax.experimental.pallas import tpu as pltpu
from jax.experimental.pallas import tpu_sc as plsc

N_NODES = 10000
N_TAB = 10240
E_RAW = 320000
E_AUG = E_RAW + N_NODES
NC, NS, LANES = 2, 16, 16
CH = 128
CPT = 164
E_PAD = NS * CPT * CH
ROWS_PER_TILE = N_TAB // NS
H = 8
HC = H // NC
HD = 16
FC = HC * HD


def _qkv_body(x_ref, wq_ref, bq_ref, wk_ref, bk_ref, wv_ref,
              q_ref, k_ref, v_ref):
    xb = x_ref[...]
    q = jnp.dot(xb, wq_ref[0], preferred_element_type=jnp.float32)
    q_ref[0] = jnp.maximum(q + bq_ref[0], 0.0)
    k = jnp.dot(xb, wk_ref[0], preferred_element_type=jnp.float32)
    k_ref[0] = jnp.maximum(k + bk_ref[0], 0.0)
    v_ref[0] = jnp.dot(xb, wv_ref[0], preferred_element_type=jnp.float32)


def _split_cols(w):
    w2 = w.reshape(w.shape[0], NC, FC) if w.ndim == 2 else w.reshape(1, NC, FC)
    return jnp.swapaxes(w2, 0, 1)


def _qkv(x_pad, wq, bq, wk, bk, wv):
    blk = 256
    grid = (N_TAB // blk, NC)
    wspec = pl.BlockSpec((1, 128, FC), lambda i, j: (j, 0, 0))
    bspec = pl.BlockSpec((1, 1, FC), lambda i, j: (j, 0, 0))
    xspec = pl.BlockSpec((blk, 128), lambda i, j: (i, 0))
    ospec = pl.BlockSpec((1, blk, FC), lambda i, j: (j, i, 0))
    out = jax.ShapeDtypeStruct((NC, N_TAB, FC), jnp.float32)
    return pl.pallas_call(
        _qkv_body,
        grid=grid,
        in_specs=[xspec, wspec, bspec, wspec, bspec, wspec],
        out_specs=[ospec, ospec, ospec],
        out_shape=[out, out, out],
    )(x_pad, _split_cols(wq), _split_cols(bq), _split_cols(wk),
      _split_cols(bk), _split_cols(wv))


def _edge_body(q_hbm, k_hbm, v_hbm, ridx_hbm, cidx_hbm, zrow_hbm, zden_hbm,
               bias_hbm,
               out_hbm,
               ridx_s, cidx_s, qb, kb, vb, eb, bias_v,
               acc_out, acc_den,
               gsem0, gsem1, isem0, isem1, isem2, isem3, ssem0, ssem1):
    c = lax.axis_index("c")
    s = lax.axis_index("s")
    lane = lax.broadcasted_iota(jnp.int32, (LANES,), 0)
    zvec = jnp.zeros((LANES,), jnp.float32)
    gsem = (gsem0, gsem1)
    isem = (isem0, isem1, isem2, isem3)
    ssem = (ssem0, ssem1)

    pltpu.sync_copy(zrow_hbm, acc_out.at[pl.ds(s * ROWS_PER_TILE, ROWS_PER_TILE)])
    pltpu.sync_copy(zden_hbm, acc_den.at[pl.ds(s * ROWS_PER_TILE, ROWS_PER_TILE)])

    pltpu.sync_copy(ridx_hbm.at[s, pl.ds(0, 4)], ridx_s)
    pltpu.sync_copy(cidx_hbm.at[s, pl.ds(0, 4)], cidx_s)

    plsc.subcore_barrier()

    def gather_copies(p, slot):
        return (
            pltpu.make_async_copy(q_hbm.at[c].at[ridx_s.at[slot]], qb.at[p],
                                  gsem[p]),
            pltpu.make_async_copy(k_hbm.at[c].at[cidx_s.at[slot]], kb.at[p],
                                  gsem[p]),
            pltpu.make_async_copy(v_hbm.at[c].at[cidx_s.at[slot]], vb.at[p],
                                  gsem[p]),
        )

    def issue_gather(p, slot):
        pltpu.async_copy(q_hbm.at[c].at[ridx_s.at[slot]], qb.at[p], gsem[p])
        pltpu.async_copy(k_hbm.at[c].at[cidx_s.at[slot]], kb.at[p], gsem[p])
        pltpu.async_copy(v_hbm.at[c].at[cidx_s.at[slot]], vb.at[p], gsem[p])

    idx15 = jnp.full((LANES, 1), HD - 1, jnp.int32)
    gdn = lax.GatherDimensionNumbers(
        offset_dims=(), collapsed_slice_dims=(0,), start_index_map=(0,))

    def _bcast_last(v):
        return lax.gather(v, idx15, gdn, slice_sizes=(1,),
                          mode=lax.GatherScatterMode.PROMISE_IN_BOUNDS)

    def _bcast_lane(v, h):
        idx = jnp.full((LANES, 1), h, jnp.int32)
        return lax.gather(v, idx, gdn, slice_sizes=(1,),
                          mode=lax.GatherScatterMode.PROMISE_IN_BOUNDS)

    def compute_chunk(p, slot):
        qbb, kbb, vbb, ebb = qb.at[p], kb.at[p], vb.at[p], eb.at[p]

        EB = 4

        def batch_body(t2, carry2):
            base = t2 * EB
            prods = []
            for i in range(EB):
                e = base + i
                for h in range(HC):
                    sl = pl.ds(h * HD, HD)
                    prods.append(qbb[e, sl] * kbb[e, sl])
            css = [plsc.cumsum(pr) for pr in prods]
            ws = [jnp.exp(_bcast_last(cs)) for cs in css]
            for i in range(EB):
                e = base + i
                row = zvec
                for h in range(HC):
                    w = ws[i * HC + h]
                    sl = pl.ds(h * HD, HD)
                    vbb[e, sl] = vbb[e, sl] * w
                    row = jnp.where(lane == h, w, row)
                ebb[e, :] = row
            return carry2

        lax.fori_loop(0, CH // EB, batch_body, 0)
        pltpu.async_copy(ebb, acc_den.at[ridx_s.at[slot]], ssem[p], add=True)
        pltpu.async_copy(vbb, acc_out.at[ridx_s.at[slot]], ssem[p], add=True)

    def drain_scatter(p, slot):
        pltpu.make_async_copy(eb.at[p], acc_den.at[ridx_s.at[slot]],
                              ssem[p]).wait()
        pltpu.make_async_copy(vb.at[p], acc_out.at[ridx_s.at[slot]],
                              ssem[p]).wait()

    issue_gather(0, 0)

    def quad_body(t, carry):
        for bb in range(4):
            j = 4 * t + bb
            p = bb % 2
            slot = bb
            pslot = (bb - 1) % 4
            nslot = (bb + 1) % 4
            for cp in gather_copies(p, slot):
                cp.wait()

            @pl.when(j >= 1)
            def _():
                drain_scatter(1 - p, pslot)

            @pl.when(jnp.logical_and(j >= 1, j + 3 < CPT))
            def _():
                pltpu.async_copy(ridx_hbm.at[s, j + 3], ridx_s.at[pslot],
                                 isem[pslot])
                pltpu.async_copy(cidx_hbm.at[s, j + 3], cidx_s.at[pslot],
                                 isem[pslot])

            nxt = j + 1

            @pl.when(jnp.logical_and(nxt >= 4, nxt < CPT))
            def _():
                pltpu.make_async_copy(ridx_hbm.at[s, nxt], ridx_s.at[nslot],
                                      isem[nslot]).wait()
                pltpu.make_async_copy(cidx_hbm.at[s, nxt], cidx_s.at[nslot],
                                      isem[nslot]).wait()

            @pl.when(nxt < CPT)
            def _():
                issue_gather(1 - p, nslot)

            compute_chunk(p, slot)
        return carry

    lax.fori_loop(0, CPT // 4, quad_body, 0)
    drain_scatter(1, 3)
    plsc.subcore_barrier()

    pltpu.sync_copy(bias_hbm.at[c], bias_v)
    bias_h = [bias_v[pl.ds(h * HD, HD)] for h in range(HC)]

    def blk_body(bi, carry):
        r0 = s * ROWS_PER_TILE + bi * CH
        pltpu.sync_copy(acc_out.at[pl.ds(r0, CH)], qb.at[0])
        pltpu.sync_copy(acc_den.at[pl.ds(r0, CH)], eb.at[0])

        def row_body(e, carry2):
            rcp = 1.0 / eb[0, e, :]
            for h in range(HC):
                sl = pl.ds(h * HD, HD)
                qb[0, e, sl] = qb[0, e, sl] * _bcast_lane(rcp, h) + bias_h[h]
            return carry2

        lax.fori_loop(0, CH, row_body, 0, unroll=2)
        pltpu.sync_copy(qb.at[0],
                        out_hbm.at[pl.ds(r0, CH), pl.ds(c * FC, FC)])
        return carry

    lax.fori_loop(0, ROWS_PER_TILE // CH, blk_body, 0)


def _edge_phase(q, k, v, ridx3, cidx3, bias2):
    mesh = plsc.VectorSubcoreMesh(core_axis_name="c", subcore_axis_name="s")
    zrow = jnp.zeros((ROWS_PER_TILE, FC), jnp.float32)
    zden = jnp.zeros((ROWS_PER_TILE, HD), jnp.float32)
    fn = pl.kernel(
        _edge_body,
        out_type=jax.ShapeDtypeStruct((N_TAB, 128), jnp.float32),
        mesh=mesh,
        compiler_params=pltpu.CompilerParams(
            needs_layout_passes=False, use_tc_tiling_on_sc=False),
        scratch_types=[
            pltpu.VMEM((4, CH), jnp.int32),
            pltpu.VMEM((4, CH), jnp.int32),
            pltpu.VMEM((2, CH, FC), jnp.float32),
            pltpu.VMEM((2, CH, FC), jnp.float32),
            pltpu.VMEM((2, CH, FC), jnp.float32),
            pltpu.VMEM((2, CH, HD), jnp.float32),
            pltpu.VMEM((FC,), jnp.float32),
            pltpu.VMEM_SHARED((N_TAB, FC), jnp.float32),
            pltpu.VMEM_SHARED((N_TAB, HD), jnp.float32),
            pltpu.SemaphoreType.DMA,
            pltpu.SemaphoreType.DMA,
            pltpu.SemaphoreType.DMA,
            pltpu.SemaphoreType.DMA,
            pltpu.SemaphoreType.DMA,
            pltpu.SemaphoreType.DMA,
            pltpu.SemaphoreType.DMA,
            pltpu.SemaphoreType.DMA,
        ],
    )
    return fn(q, k, v, ridx3, cidx3, zrow, zden, bias2)


@jax.jit
def kernel(x, edge_index, query_kernel, query_bias, key_kernel, key_bias,
           kernel, bias):
    n = x.shape[0]
    x_pad = jnp.concatenate(
        [x, jnp.zeros((N_TAB - n, x.shape[1]), x.dtype)], axis=0)
    q, k, v = _qkv(x_pad, query_kernel, query_bias, key_kernel, key_bias,
                   kernel)

    self_loop = jnp.arange(n, dtype=edge_index.dtype)
    rows = jnp.concatenate([edge_index[0], self_loop])
    cols = jnp.concatenate([edge_index[1], self_loop])
    pad = E_PAD - E_AUG
    dummy = jnp.full((pad,), N_NODES, dtype=rows.dtype)
    ridx3 = jnp.concatenate([rows, dummy]).reshape(NS, CPT, CH)
    cidx3 = jnp.concatenate([cols, dummy]).reshape(NS, CPT, CH)

    bias2 = bias.reshape(NC, FC)
    outp = _edge_phase(q, k, v, ridx3, cidx3, bias2)
    return outp[:N_NODES]
